# Initial kernel scaffold; baseline (speedup 1.0000x reference)
import jax
import jax.numpy as jnp
from jax.experimental import pallas as pl


def kernel(max_pred_types, instrument_scores):
    raise NotImplementedError("write your pallas kernel here")


# speedup vs baseline: 2.1478x; 2.1478x over previous
import functools

import jax
import jax.numpy as jnp
from jax import lax
from jax.experimental import pallas as pl
from jax.experimental.pallas import tpu as pltpu
from jax.experimental.pallas import tpu_sc as plsc

_B, _T, _I = 8, 8192, 256
_BT = 1024
_L = 16


def _argmax_body(scores_ref, out_ref):
    x = scores_ref[0]
    m = jnp.max(x, axis=1, keepdims=True)
    iota = lax.broadcasted_iota(jnp.int32, (_BT, _I), 1)
    cand = jnp.where(x == m, iota, _I)
    out_ref[0, 0, :] = jnp.min(cand, axis=1)


def _tc_argmax(scores, interpret=False):
    nblk = _T // _BT
    out = pl.pallas_call(
        _argmax_body,
        grid=(_B, nblk),
        in_specs=[pl.BlockSpec((1, _BT, _I), lambda b, t: (b, t, 0))],
        out_specs=pl.BlockSpec((1, 1, _BT), lambda b, t: (b * nblk + t, 0, 0)),
        out_shape=jax.ShapeDtypeStruct((_B * nblk, 1, _BT), jnp.int32),
        interpret=interpret,
    )(scores)
    return out.reshape(_B, _T)


def _sc_body(types_hbm, inst_hbm, partials_hbm, total_hbm,
             types_v, inst_v, pres_v, stage_v, red_v):
    c = lax.axis_index("c")
    s = lax.axis_index("s")
    is_worker = jnp.logical_and(c == 0, s < _B)

    @pl.when(is_worker)
    def _():
        b = s
        pltpu.sync_copy(types_hbm.at[b], types_v)
        pltpu.sync_copy(inst_hbm.at[b], inst_v)

        zeros = jnp.zeros((_L,), jnp.int32)
        for i in range(_I // _L):
            pres_v[pl.ds(i * _L, _L)] = zeros
        ones = jnp.ones((_L,), jnp.int32)

        def pass_scatter(t, n1_acc):
            tv = types_v[pl.ds(t * _L, _L)]
            iv = inst_v[pl.ds(t * _L, _L)]
            m1 = tv == 1
            plsc.store_scatter(pres_v, [iv], ones, mask=m1)
            return n1_acc + jnp.where(m1, 1, 0).astype(jnp.int32)

        n1 = lax.fori_loop(0, _T // _L, pass_scatter, zeros)

        u1 = jnp.zeros((_L,), jnp.int32)
        for i in range(_I // _L):
            u1 = u1 + pres_v[pl.ds(i * _L, _L)]

        def pass_gather(t, r_acc):
            tv = types_v[pl.ds(t * _L, _L)]
            iv = inst_v[pl.ds(t * _L, _L)]
            g = plsc.load_gather(pres_v, [iv])
            miss = jnp.logical_and(tv == 3, g == 0)
            return r_acc + jnp.where(miss, 1, 0).astype(jnp.int32)

        r22 = lax.fori_loop(0, _T // _L, pass_gather, zeros)

        stage_v[...] = n1 - u1 + r22
        pltpu.sync_copy(stage_v, partials_hbm.at[pl.ds(b * _L, _L)])

    plsc.subcore_barrier()

    @pl.when(jnp.logical_and(c == 0, s == 0))
    def _():
        pltpu.sync_copy(partials_hbm, red_v)
        acc = jnp.zeros((_L,), jnp.int32)
        for i in range(_B):
            acc = acc + red_v[pl.ds(i * _L, _L)]
        total = jnp.sum(acc)
        stage_v[...] = jnp.full((_L,), total, jnp.int32)
        pltpu.sync_copy(stage_v, total_hbm)


@functools.cache
def _sc_binning():
    return pl.kernel(
        _sc_body,
        out_type=(
            jax.ShapeDtypeStruct((_B * _L,), jnp.int32),
            jax.ShapeDtypeStruct((_L,), jnp.int32),
        ),
        mesh=plsc.VectorSubcoreMesh(core_axis_name="c", subcore_axis_name="s"),
        compiler_params=pltpu.CompilerParams(needs_layout_passes=False),
        scratch_types=[
            pltpu.VMEM((_T,), jnp.int32),
            pltpu.VMEM((_T,), jnp.int32),
            pltpu.VMEM((_I,), jnp.int32),
            pltpu.VMEM((_L,), jnp.int32),
            pltpu.VMEM((_B * _L,), jnp.int32),
        ],
    )


def kernel(max_pred_types, instrument_scores):
    inst = _tc_argmax(instrument_scores)
    _, total_vec = _sc_binning()(max_pred_types, inst)
    return total_vec[0]
